# Initial kernel scaffold; baseline (speedup 1.0000x reference)
import jax
import jax.numpy as jnp
from jax.experimental import pallas as pl


def kernel(x, W_router, W1, b1, W2, b2):
    raise NotImplementedError("write your pallas kernel here")


# speedup vs baseline: 1.4590x; 1.4590x over previous
import functools

import jax
import jax.numpy as jnp
from jax import lax
from jax.experimental import pallas as pl
from jax.experimental.pallas import tpu as pltpu
from jax.experimental.pallas import tpu_sc as plsc

D = 768
F = 3072
E = 8
T = 2048
LBW = 0.01
M = 256
NPAIR = 2 * T
NPAD = NPAIR + E * M
NT = NPAD // M
NMETA = 128

SC_CORES = 2
SC_SUBCORES = 16
NW = SC_CORES * SC_SUBCORES
CHUNK = NPAIR // NW

_CB = 128


def _router_body(x_ref, wr_ref, dst_ref, cw_ref, meta_ref, aux_ref, rank_ref,
                 sel_ref):
    x = x_ref[...]
    logits = jnp.dot(x, wr_ref[...], preferred_element_type=jnp.float32)
    mx = jnp.max(logits, axis=1, keepdims=True)
    ex = jnp.exp(logits - mx)
    probs = ex / jnp.sum(ex, axis=1, keepdims=True)

    eidx = lax.broadcasted_iota(jnp.int32, (T, E), 1)
    p1 = jnp.max(probs, axis=1, keepdims=True)
    i1 = jnp.min(jnp.where(probs == p1, eidx, E), axis=1, keepdims=True)
    probs2 = jnp.where(eidx == i1, -jnp.inf, probs)
    p2 = jnp.max(probs2, axis=1, keepdims=True)
    i2 = jnp.min(jnp.where(probs2 == p2, eidx, E), axis=1, keepdims=True)
    s = p1 + p2
    cw_ref[...] = jnp.concatenate([p1 / s, p2 / s], axis=1)

    sel_ref[...] = ((eidx == i1) | (eidx == i2)).astype(jnp.float32)

    ri = lax.broadcasted_iota(jnp.int32, (_CB, _CB), 0)
    ci = lax.broadcasted_iota(jnp.int32, (_CB, _CB), 1)
    ltri = (ci < ri).astype(jnp.bfloat16)

    def chunk_step(c, base):
        sc = sel_ref[pl.ds(c * _CB, _CB), :].astype(jnp.bfloat16)
        rank_ref[pl.ds(c * _CB, _CB), :] = (
            jnp.dot(ltri, sc, preferred_element_type=jnp.float32) + base)
        return base + jnp.sum(sc.astype(jnp.float32), axis=0, keepdims=True)

    counts = lax.fori_loop(0, T // _CB, chunk_step,
                           jnp.zeros((1, E), jnp.float32))

    ci8 = counts.astype(jnp.int32)
    padded = ((ci8 + (M - 1)) // M) * M
    pf = padded.astype(jnp.float32)
    ai = lax.broadcasted_iota(jnp.int32, (E, E), 0)
    bi = lax.broadcasted_iota(jnp.int32, (E, E), 1)
    off = jnp.dot(pf, (ai < bi).astype(jnp.float32),
                  preferred_element_type=jnp.float32)
    end = off + pf
    total = jnp.sum(padded)

    posmat = off + rank_ref[...]
    d1 = jnp.sum(jnp.where(eidx == i1, posmat, 0.0), axis=1, keepdims=True)
    d2 = jnp.sum(jnp.where(eidx == i2, posmat, 0.0), axis=1, keepdims=True)
    dst_ref[...] = jnp.concatenate([d1, d2], axis=1).astype(jnp.int32)

    tstart = lax.broadcasted_iota(jnp.int32, (1, NMETA), 1) * M
    cs = jnp.minimum(tstart, total - M)
    endi = end.astype(jnp.int32)
    g = jnp.zeros((1, NMETA), jnp.int32)
    for e in range(E):
        g = g + (endi[0:1, e:e + 1] <= cs).astype(jnp.int32)
    meta_ref[0:1, :] = g
    meta_ref[1:2, :] = jnp.full((1, NMETA), total // M, jnp.int32)

    usage = jnp.mean(probs, axis=0, keepdims=True)
    aux_ref[...] = jnp.sum((usage - 1.0 / E) ** 2,
                           axis=1, keepdims=True) * LBW


def _router(x2, w_router):
    return pl.pallas_call(
        _router_body,
        out_shape=(
            jax.ShapeDtypeStruct((T, 2), jnp.int32),
            jax.ShapeDtypeStruct((T, 2), jnp.float32),
            jax.ShapeDtypeStruct((2, NMETA), jnp.int32),
            jax.ShapeDtypeStruct((1, 1), jnp.float32),
        ),
        scratch_shapes=[pltpu.VMEM((T, E), jnp.float32),
                        pltpu.VMEM((T, E), jnp.float32)],
    )(x2, w_router)


@functools.cache
def _sc_kernels():
    mesh = plsc.VectorSubcoreMesh(
        core_axis_name="c", subcore_axis_name="s",
        num_cores=SC_CORES, num_subcores=SC_SUBCORES)
    scratch = [
        pltpu.VMEM((CHUNK,), jnp.int32),
        pltpu.VMEM((CHUNK, D), jnp.float32),
        pltpu.SemaphoreType.DMA,
    ]

    @functools.partial(
        pl.kernel,
        out_type=jax.ShapeDtypeStruct((NPAD, D), jnp.float32),
        mesh=mesh, scratch_types=scratch)
    def sc_scatter(x_hbm, idx_hbm, xs_hbm, idx_v, rows_v, sem):
        wid = lax.axis_index("s") * SC_CORES + lax.axis_index("c")
        base = wid * CHUNK
        src = lax.rem(base, T)
        pltpu.sync_copy(idx_hbm.at[wid], idx_v)
        pltpu.sync_copy(x_hbm.at[pl.ds(src, CHUNK)], rows_v)
        pltpu.async_copy(rows_v, xs_hbm.at[idx_v], sem).wait()

    @functools.partial(
        pl.kernel,
        out_type=jax.ShapeDtypeStruct((NPAIR, D), jnp.float32),
        mesh=mesh, scratch_types=scratch)
    def sc_gather(ys_hbm, idx_hbm, g_hbm, idx_v, rows_v, sem):
        wid = lax.axis_index("s") * SC_CORES + lax.axis_index("c")
        base = wid * CHUNK
        pltpu.sync_copy(idx_hbm.at[wid], idx_v)
        pltpu.async_copy(ys_hbm.at[idx_v], rows_v, sem).wait()
        pltpu.sync_copy(rows_v, g_hbm.at[pl.ds(base, CHUNK)])

    return sc_scatter, sc_gather


def _ffn_body(meta_ref, xs_ref, w1_ref, b1_ref, w2_ref, b2_ref, ys_ref):
    i = pl.program_id(0)

    @pl.when(i < meta_ref[1, 0])
    def _():
        h = jnp.dot(xs_ref[...], w1_ref[0], preferred_element_type=jnp.float32)
        h = jnp.maximum(h + b1_ref[0], 0.0)
        y = jnp.dot(h, w2_ref[0], preferred_element_type=jnp.float32)
        ys_ref[...] = y + b2_ref[0]


def _ffn(meta, xs, w1, b1, w2, b2):
    grid_spec = pltpu.PrefetchScalarGridSpec(
        num_scalar_prefetch=1,
        grid=(NT,),
        in_specs=[
            pl.BlockSpec((M, D), lambda i, m: (i, 0)),
            pl.BlockSpec((1, D, F), lambda i, m: (m[0, i], 0, 0)),
            pl.BlockSpec((1, 1, F), lambda i, m: (m[0, i], 0, 0)),
            pl.BlockSpec((1, F, D), lambda i, m: (m[0, i], 0, 0)),
            pl.BlockSpec((1, 1, D), lambda i, m: (m[0, i], 0, 0)),
        ],
        out_specs=pl.BlockSpec((M, D), lambda i, m: (i, 0)),
    )
    return pl.pallas_call(
        _ffn_body,
        grid_spec=grid_spec,
        out_shape=jax.ShapeDtypeStruct((NPAD, D), jnp.float32),
    )(meta, xs, w1, b1.reshape(E, 1, F), w2, b2.reshape(E, 1, D))


def _combine_body(g1_ref, g2_ref, cw_ref, out_ref):
    out_ref[...] = (cw_ref[:, 0:1] * g1_ref[...] +
                    cw_ref[:, 1:2] * g2_ref[...])


def _combine(g, cw):
    nb = T // M
    return pl.pallas_call(
        _combine_body,
        grid=(nb,),
        in_specs=[
            pl.BlockSpec((M, D), lambda i: (i, 0)),
            pl.BlockSpec((M, D), lambda i: (i + nb, 0)),
            pl.BlockSpec((M, 2), lambda i: (i, 0)),
        ],
        out_specs=pl.BlockSpec((M, D), lambda i: (i, 0)),
        out_shape=jax.ShapeDtypeStruct((T, D), jnp.float32),
    )(g, g, cw)


def kernel(x, W_router, W1, b1, W2, b2):
    x2 = x.reshape(T, D)
    dst, cw, meta, aux = _router(x2, W_router)
    idx = dst.T.reshape(NW, CHUNK)
    sc_scatter, sc_gather = _sc_kernels()
    xs = sc_scatter(x2, idx)
    ys = _ffn(meta, xs, W1, b1, W2, b2)
    g = sc_gather(ys, idx)
    out = _combine(g, cw)
    return out.reshape(1, T, D), aux[0, 0]


# speedup vs baseline: 1.4623x; 1.0022x over previous
import functools

import jax
import jax.numpy as jnp
from jax import lax
from jax.experimental import pallas as pl
from jax.experimental.pallas import tpu as pltpu
from jax.experimental.pallas import tpu_sc as plsc

D = 768
F = 3072
E = 8
T = 2048
LBW = 0.01
M = 256
NPAIR = 2 * T
NPAD = NPAIR + E * M
NT = NPAD // M
NMETA = 128

SC_CORES = 2
SC_SUBCORES = 16
NW = SC_CORES * SC_SUBCORES
CHUNK = NPAIR // NW

_CB = 128


def _router_body(x_ref, wr_ref, dst_ref, cw_ref, meta_ref, aux_ref, rank_ref,
                 sel_ref):
    x = x_ref[...]
    logits = jnp.dot(x, wr_ref[...], preferred_element_type=jnp.float32)
    mx = jnp.max(logits, axis=1, keepdims=True)
    ex = jnp.exp(logits - mx)
    probs = ex / jnp.sum(ex, axis=1, keepdims=True)

    eidx = lax.broadcasted_iota(jnp.int32, (T, E), 1)
    p1 = jnp.max(probs, axis=1, keepdims=True)
    i1 = jnp.min(jnp.where(probs == p1, eidx, E), axis=1, keepdims=True)
    probs2 = jnp.where(eidx == i1, -jnp.inf, probs)
    p2 = jnp.max(probs2, axis=1, keepdims=True)
    i2 = jnp.min(jnp.where(probs2 == p2, eidx, E), axis=1, keepdims=True)
    s = p1 + p2
    cw_ref[...] = jnp.concatenate([p1 / s, p2 / s], axis=1)

    sel_ref[...] = ((eidx == i1) | (eidx == i2)).astype(jnp.float32)

    ri = lax.broadcasted_iota(jnp.int32, (_CB, _CB), 0)
    ci = lax.broadcasted_iota(jnp.int32, (_CB, _CB), 1)
    ltri = (ci < ri).astype(jnp.bfloat16)

    def chunk_step(c, base):
        sc = sel_ref[pl.ds(c * _CB, _CB), :].astype(jnp.bfloat16)
        rank_ref[pl.ds(c * _CB, _CB), :] = (
            jnp.dot(ltri, sc, preferred_element_type=jnp.float32) + base)
        return base + jnp.sum(sc.astype(jnp.float32), axis=0, keepdims=True)

    counts = lax.fori_loop(0, T // _CB, chunk_step,
                           jnp.zeros((1, E), jnp.float32))

    ci8 = counts.astype(jnp.int32)
    padded = ((ci8 + (M - 1)) // M) * M
    pf = padded.astype(jnp.float32)
    ai = lax.broadcasted_iota(jnp.int32, (E, E), 0)
    bi = lax.broadcasted_iota(jnp.int32, (E, E), 1)
    off = jnp.dot(pf, (ai < bi).astype(jnp.float32),
                  preferred_element_type=jnp.float32)
    end = off + pf
    total = jnp.sum(padded)

    posmat = off + rank_ref[...]
    d1 = jnp.sum(jnp.where(eidx == i1, posmat, 0.0), axis=1, keepdims=True)
    d2 = jnp.sum(jnp.where(eidx == i2, posmat, 0.0), axis=1, keepdims=True)
    dst_ref[...] = jnp.concatenate([d1, d2], axis=1).astype(jnp.int32)

    tstart = lax.broadcasted_iota(jnp.int32, (1, NMETA), 1) * M
    cs = jnp.minimum(tstart, total - M)
    endi = end.astype(jnp.int32)
    g = jnp.zeros((1, NMETA), jnp.int32)
    for e in range(E):
        g = g + (endi[0:1, e:e + 1] <= cs).astype(jnp.int32)
    meta_ref[0:1, :] = g
    meta_ref[1:2, :] = jnp.full((1, NMETA), total // M, jnp.int32)

    usage = jnp.mean(probs, axis=0, keepdims=True)
    aux_ref[...] = jnp.sum((usage - 1.0 / E) ** 2,
                           axis=1, keepdims=True) * LBW


def _router(x2, w_router):
    return pl.pallas_call(
        _router_body,
        out_shape=(
            jax.ShapeDtypeStruct((T, 2), jnp.int32),
            jax.ShapeDtypeStruct((T, 2), jnp.float32),
            jax.ShapeDtypeStruct((2, NMETA), jnp.int32),
            jax.ShapeDtypeStruct((1, 1), jnp.float32),
        ),
        scratch_shapes=[pltpu.VMEM((T, E), jnp.float32),
                        pltpu.VMEM((T, E), jnp.float32)],
    )(x2, w_router)


@functools.cache
def _sc_kernels():
    mesh = plsc.VectorSubcoreMesh(
        core_axis_name="c", subcore_axis_name="s",
        num_cores=SC_CORES, num_subcores=SC_SUBCORES)
    scratch = [
        pltpu.VMEM((CHUNK,), jnp.int32),
        pltpu.VMEM((CHUNK, D), jnp.float32),
        pltpu.SemaphoreType.DMA,
    ]

    @functools.partial(
        pl.kernel,
        out_type=jax.ShapeDtypeStruct((NPAD, D), jnp.float32),
        mesh=mesh, scratch_types=scratch)
    def sc_scatter(x_hbm, idx_hbm, xs_hbm, idx_v, rows_v, sem):
        wid = lax.axis_index("s") * SC_CORES + lax.axis_index("c")
        base = wid * CHUNK
        src = lax.rem(base, T)
        pltpu.sync_copy(idx_hbm.at[wid], idx_v)
        pltpu.sync_copy(x_hbm.at[pl.ds(src, CHUNK)], rows_v)
        pltpu.async_copy(rows_v, xs_hbm.at[idx_v], sem).wait()

    @functools.partial(
        pl.kernel,
        out_type=jax.ShapeDtypeStruct((NPAIR, D), jnp.float32),
        mesh=mesh, scratch_types=scratch)
    def sc_gather(ys_hbm, idx_hbm, g_hbm, idx_v, rows_v, sem):
        wid = lax.axis_index("s") * SC_CORES + lax.axis_index("c")
        base = wid * CHUNK
        pltpu.sync_copy(idx_hbm.at[wid], idx_v)
        pltpu.async_copy(ys_hbm.at[idx_v], rows_v, sem).wait()
        pltpu.sync_copy(rows_v, g_hbm.at[pl.ds(base, CHUNK)])

    return sc_scatter, sc_gather


def _ffn_body(meta_ref, xs_ref, w1_ref, b1_ref, w2_ref, b2_ref, ys_ref):
    i = pl.program_id(0)

    @pl.when(i < meta_ref[1, 0])
    def _():
        h = jnp.dot(xs_ref[...], w1_ref[0], preferred_element_type=jnp.float32,
                    precision=lax.Precision.DEFAULT)
        h = jnp.maximum(h + b1_ref[0], 0.0)
        y = jnp.dot(h, w2_ref[0], preferred_element_type=jnp.float32,
                    precision=lax.Precision.DEFAULT)
        ys_ref[...] = y + b2_ref[0]


def _ffn(meta, xs, w1, b1, w2, b2):
    grid_spec = pltpu.PrefetchScalarGridSpec(
        num_scalar_prefetch=1,
        grid=(NT,),
        in_specs=[
            pl.BlockSpec((M, D), lambda i, m: (i, 0)),
            pl.BlockSpec((1, D, F), lambda i, m: (m[0, i], 0, 0)),
            pl.BlockSpec((1, 1, F), lambda i, m: (m[0, i], 0, 0)),
            pl.BlockSpec((1, F, D), lambda i, m: (m[0, i], 0, 0)),
            pl.BlockSpec((1, 1, D), lambda i, m: (m[0, i], 0, 0)),
        ],
        out_specs=pl.BlockSpec((M, D), lambda i, m: (i, 0)),
    )
    return pl.pallas_call(
        _ffn_body,
        grid_spec=grid_spec,
        out_shape=jax.ShapeDtypeStruct((NPAD, D), jnp.float32),
    )(meta, xs, w1, b1.reshape(E, 1, F), w2, b2.reshape(E, 1, D))


def _combine_body(g1_ref, g2_ref, cw_ref, out_ref):
    out_ref[...] = (cw_ref[:, 0:1] * g1_ref[...] +
                    cw_ref[:, 1:2] * g2_ref[...])


def _combine(g, cw):
    nb = T // M
    return pl.pallas_call(
        _combine_body,
        grid=(nb,),
        in_specs=[
            pl.BlockSpec((M, D), lambda i: (i, 0)),
            pl.BlockSpec((M, D), lambda i: (i + nb, 0)),
            pl.BlockSpec((M, 2), lambda i: (i, 0)),
        ],
        out_specs=pl.BlockSpec((M, D), lambda i: (i, 0)),
        out_shape=jax.ShapeDtypeStruct((T, D), jnp.float32),
    )(g, g, cw)


def kernel(x, W_router, W1, b1, W2, b2):
    x2 = x.reshape(T, D)
    dst, cw, meta, aux = _router(x2, W_router)
    idx = dst.T.reshape(NW, CHUNK)
    sc_scatter, sc_gather = _sc_kernels()
    xs = sc_scatter(x2, idx)
    ys = _ffn(meta, xs, W1, b1, W2, b2)
    g = sc_gather(ys, idx)
    out = _combine(g, cw)
    return out.reshape(1, T, D), aux[0, 0]
